# baseline (speedup 1.0000x reference)
import functools

import jax
import jax.numpy as jnp
from jax.experimental import pallas as pl
from jax.experimental.pallas import tpu as pltpu

_DILATIONS = (12, 24, 36)


def _fused_kernel(xp_ref, xq_ref, b0w_ref, b0o_ref, dilw_ref, dilo_ref,
                  poolw_ref, poolo_ref, projw_ref, projo_ref,
                  headw_ref, heado_ref, clsw_ref, clsb_ref,
                  o_ref, pacc, conv, pbuf, *, H, W, P, Q, dils):
    cin = xp_ref.shape[-1]
    C = b0w_ref.shape[-1]
    CP = clsw_ref.shape[-1]
    HW = H * W
    f32 = jnp.float32
    bf16 = jnp.bfloat16

    interior = xq_ref[0, P:P + H, Q:Q + W, :].reshape(HW, cin)

    mean = jnp.mean(interior.astype(f32), axis=0, keepdims=True)
    pooled = jnp.dot(mean.astype(bf16), poolw_ref[...],
                     preferred_element_type=f32)
    pooled = jnp.maximum(pooled + poolo_ref[...], 0.0)
    pool_proj = jnp.dot(pooled.astype(bf16), projw_ref[4],
                        preferred_element_type=f32)

    b0 = jnp.dot(interior, b0w_ref[...], preferred_element_type=f32)
    b0 = jnp.maximum(b0 + b0o_ref[...], 0.0)
    pacc[...] = (jnp.dot(b0.astype(bf16), projw_ref[0],
                         preferred_element_type=f32) + pool_proj)

    for i, d in enumerate(dils):
        first = True
        for kh in (1, 0, 2):
            dh = (kh - 1) * d
            lo = max(0, -dh)
            hi = H - max(0, dh)
            if lo >= hi:
                continue
            for kw in range(3):
                dw = (kw - 1) * d
                if (P + dw) % 8 == 0:
                    patch = xp_ref[0, P + dh + lo:P + dh + hi,
                                   P + dw:P + dw + W, :]
                else:
                    patch = xq_ref[0, P + dh + lo:P + dh + hi,
                                   Q + dw:Q + dw + W, :]
                patch = patch.reshape((hi - lo) * W, cin)
                contrib = jnp.dot(patch, dilw_ref[i * 9 + kh * 3 + kw],
                                  preferred_element_type=f32)
                if first:
                    conv[...] = contrib
                    first = False
                else:
                    conv[lo * W:hi * W, :] += contrib
        bi = jnp.maximum(conv[...] + dilo_ref[i], 0.0)
        pacc[...] += jnp.dot(bi.astype(bf16), projw_ref[i + 1],
                             preferred_element_type=f32)

    proj = jnp.maximum(pacc[...] + projo_ref[...], 0.0).astype(bf16)
    pbuf[0:1, :, :] = jnp.zeros((1, W + 2, C), bf16)
    pbuf[H + 1:H + 2, :, :] = jnp.zeros((1, W + 2, C), bf16)
    pbuf[:, 0:1, :] = jnp.zeros((H + 2, 1, C), bf16)
    pbuf[:, W + 1:W + 2, :] = jnp.zeros((H + 2, 1, C), bf16)
    pbuf[1:H + 1, 1:W + 1, :] = proj.reshape(H, W, C)

    for kh in range(3):
        for kw in range(3):
            patch = pbuf[kh:kh + H, kw:kw + W, :].reshape(HW, C)
            contrib = jnp.dot(patch, headw_ref[kh * 3 + kw],
                              preferred_element_type=f32)
            if kh == 0 and kw == 0:
                conv[...] = contrib
            else:
                conv[...] += contrib
    h = jnp.maximum(conv[...] + heado_ref[...], 0.0).astype(bf16)

    logits = jax.lax.dot_general(clsw_ref[...], h,
                                 (((0,), (1,)), ((), ())),
                                 preferred_element_type=f32)
    o_ref[...] = (logits + clsb_ref[...]).reshape(1, CP, HW)


def kernel(b0_w, b0_scale, b0_offset, dil_w, dil_scale, dil_offset,
           pool_w, pool_scale, pool_offset, proj_w, proj_scale, proj_offset,
           head_w, head_scale, head_offset, cls_w, cls_b, x):
    N, cin, H, W = x.shape
    C = b0_w.shape[-1]
    P = max(_DILATIONS)
    nc = cls_w.shape[1]
    CP = max(32, ((nc + 7) // 8) * 8)
    HW = H * W
    bf = jnp.bfloat16

    Q = 24
    xh = jnp.transpose(x, (0, 2, 3, 1)).astype(bf)
    xp = jnp.pad(xh, ((0, 0), (P, P), (P, P), (0, 0)))
    xq = jnp.pad(xh, ((0, 0), (P, P), (Q, Q), (0, 0)))
    Hp, Wp = H + 2 * P, W + 2 * P
    Wq = W + 2 * Q

    b0w = (b0_w * b0_scale).astype(bf)
    dilw = (dil_w * dil_scale[:, None, None]).reshape(9 * len(_DILATIONS),
                                                     cin, C).astype(bf)
    poolw = (pool_w * pool_scale).astype(bf)
    projw = (proj_w * proj_scale).astype(bf)
    headw = (head_w * head_scale).reshape(9, C, C).astype(bf)
    clsw = jnp.pad(cls_w, ((0, 0), (0, CP - nc))).astype(bf)
    clsb = jnp.pad(cls_b, ((0, 0), (0, CP - nc))).reshape(CP, 1)

    def const(*shape):
        nd = len(shape)
        return pl.BlockSpec(shape, lambda n, _nd=nd: (0,) * _nd)

    out = pl.pallas_call(
        functools.partial(_fused_kernel, H=H, W=W, P=P, Q=Q, dils=_DILATIONS),
        out_shape=jax.ShapeDtypeStruct((N, CP, HW), jnp.float32),
        grid=(N,),
        in_specs=[
            pl.BlockSpec((1, Hp, Wp, cin), lambda n: (n, 0, 0, 0)),
            pl.BlockSpec((1, Hp, Wq, cin), lambda n: (n, 0, 0, 0)),
            const(cin, C), const(1, C),
            const(9 * len(_DILATIONS), cin, C), const(len(_DILATIONS), 1, C),
            const(cin, C), const(1, C),
            const(5, C, C), const(1, C),
            const(9, C, C), const(1, C),
            const(cin, CP), const(CP, 1),
        ],
        out_specs=pl.BlockSpec((1, CP, HW), lambda n: (n, 0, 0)),
        scratch_shapes=[
            pltpu.VMEM((HW, C), jnp.float32),
            pltpu.VMEM((HW, C), jnp.float32),
            pltpu.VMEM((H + 2, W + 2, C), bf),
        ],
        compiler_params=pltpu.CompilerParams(
            dimension_semantics=("parallel",),
            vmem_limit_bytes=110 * 1024 * 1024),
    )(xp, xq, b0w, b0_offset, dilw, dil_offset, poolw, pool_offset,
      projw, proj_offset, headw, head_offset, clsw, clsb)
    return out[:, :nc, :].reshape(N, nc, H, W)


# speedup vs baseline: 1.1124x; 1.1124x over previous
import functools

import jax
import jax.numpy as jnp
from jax.experimental import pallas as pl
from jax.experimental.pallas import tpu as pltpu

_DILATIONS = (12, 24, 36)
_CH = 8


def _fused_kernel(xp_ref, b0w_ref, b0o_ref, dilw_ref, dilo_ref,
                  poolw_ref, poolo_ref, projw_ref, projo_ref,
                  headw_ref, heado_ref, clsw_ref, clsb_ref,
                  o_ref, pbuf, *, H, W, P, dils):
    cin = xp_ref.shape[-1]
    C = b0w_ref.shape[-1]
    CP = clsw_ref.shape[-1]
    f32 = jnp.float32
    bf16 = jnp.bfloat16
    CH = _CH
    M = CH * W

    interior = xp_ref[0, P:P + H, P:P + W, :].reshape(H * W, cin)
    mean = jnp.mean(interior.astype(f32), axis=0, keepdims=True)
    pooled = jnp.dot(mean.astype(bf16), poolw_ref[...],
                     preferred_element_type=f32)
    pooled = jnp.maximum(pooled + poolo_ref[...], 0.0)
    pool_proj = jnp.dot(pooled.astype(bf16), projw_ref[4],
                        preferred_element_type=f32)

    pbuf[0:1, :, :] = jnp.zeros((1, W + 2, C), bf16)
    pbuf[H + 1:H + 2, :, :] = jnp.zeros((1, W + 2, C), bf16)
    pbuf[:, 0:1, :] = jnp.zeros((H + 2, 1, C), bf16)
    pbuf[:, W + 1:W + 2, :] = jnp.zeros((H + 2, 1, C), bf16)

    for c in range(H // CH):
        r0 = c * CH
        xs = xp_ref[0, P + r0:P + r0 + CH, P:P + W, :].reshape(M, cin)
        b0 = jnp.dot(xs, b0w_ref[...], preferred_element_type=f32)
        b0 = jnp.maximum(b0 + b0o_ref[...], 0.0)
        pacc = jnp.dot(b0.astype(bf16), projw_ref[0],
                       preferred_element_type=f32) + pool_proj
        for i, d in enumerate(dils):
            conv = None
            for kh in range(3):
                dh = (kh - 1) * d
                lo = max(0, -dh)
                hi = H - max(0, dh)
                if r0 + CH <= lo or r0 >= hi:
                    continue
                for kw in range(3):
                    dw = (kw - 1) * d
                    patch = xp_ref[0, P + dh + r0:P + dh + r0 + CH,
                                   P + dw:P + dw + W, :].reshape(M, cin)
                    t = jnp.dot(patch, dilw_ref[i * 9 + kh * 3 + kw],
                                preferred_element_type=f32)
                    conv = t if conv is None else conv + t
            bi = jnp.maximum(conv + dilo_ref[i], 0.0)
            pacc = pacc + jnp.dot(bi.astype(bf16), projw_ref[i + 1],
                                  preferred_element_type=f32)
        proj = jnp.maximum(pacc + projo_ref[...], 0.0).astype(bf16)
        pbuf[1 + r0:1 + r0 + CH, 1:W + 1, :] = proj.reshape(CH, W, C)

    for c in range(H // CH):
        r0 = c * CH
        hacc = None
        for kh in range(3):
            for kw in range(3):
                patch = pbuf[r0 + kh:r0 + kh + CH, kw:kw + W, :].reshape(M, C)
                t = jnp.dot(patch, headw_ref[kh * 3 + kw],
                            preferred_element_type=f32)
                hacc = t if hacc is None else hacc + t
        h = jnp.maximum(hacc + heado_ref[...], 0.0).astype(bf16)
        logits = jax.lax.dot_general(clsw_ref[...], h,
                                     (((0,), (1,)), ((), ())),
                                     preferred_element_type=f32)
        o_ref[0, :, r0 * W:r0 * W + M] = logits + clsb_ref[...]


def kernel(b0_w, b0_scale, b0_offset, dil_w, dil_scale, dil_offset,
           pool_w, pool_scale, pool_offset, proj_w, proj_scale, proj_offset,
           head_w, head_scale, head_offset, cls_w, cls_b, x):
    N, cin, H, W = x.shape
    C = b0_w.shape[-1]
    P = max(_DILATIONS)
    nc = cls_w.shape[1]
    CP = max(32, ((nc + 7) // 8) * 8)
    HW = H * W
    bf = jnp.bfloat16

    xh = jnp.transpose(x, (0, 2, 3, 1)).astype(bf)
    xp = jnp.pad(xh, ((0, 0), (P, P), (P, P), (0, 0)))
    Hp, Wp = H + 2 * P, W + 2 * P

    b0w = (b0_w * b0_scale).astype(bf)
    dilw = (dil_w * dil_scale[:, None, None]).reshape(9 * len(_DILATIONS),
                                                     cin, C).astype(bf)
    poolw = (pool_w * pool_scale).astype(bf)
    projw = (proj_w * proj_scale).astype(bf)
    headw = (head_w * head_scale).reshape(9, C, C).astype(bf)
    clsw = jnp.pad(cls_w, ((0, 0), (0, CP - nc))).astype(bf)
    clsb = jnp.pad(cls_b, ((0, 0), (0, CP - nc))).reshape(CP, 1)

    def const(*shape):
        nd = len(shape)
        return pl.BlockSpec(shape, lambda n, _nd=nd: (0,) * _nd)

    out = pl.pallas_call(
        functools.partial(_fused_kernel, H=H, W=W, P=P, dils=_DILATIONS),
        out_shape=jax.ShapeDtypeStruct((N, CP, HW), jnp.float32),
        grid=(N,),
        in_specs=[
            pl.BlockSpec((1, Hp, Wp, cin), lambda n: (n, 0, 0, 0)),
            const(cin, C), const(1, C),
            const(9 * len(_DILATIONS), cin, C), const(len(_DILATIONS), 1, C),
            const(cin, C), const(1, C),
            const(5, C, C), const(1, C),
            const(9, C, C), const(1, C),
            const(cin, CP), const(CP, 1),
        ],
        out_specs=pl.BlockSpec((1, CP, HW), lambda n: (n, 0, 0)),
        scratch_shapes=[
            pltpu.VMEM((H + 2, W + 2, C), bf),
        ],
        compiler_params=pltpu.CompilerParams(
            dimension_semantics=("parallel",),
            vmem_limit_bytes=100 * 1024 * 1024),
    )(xp, b0w, b0_offset, dilw, dil_offset, poolw, pool_offset,
      projw, proj_offset, headw, head_offset, clsw, clsb)
    return out[:, :nc, :].reshape(N, nc, H, W)


# speedup vs baseline: 1.1356x; 1.0208x over previous
import functools

import jax
import jax.numpy as jnp
from jax.experimental import pallas as pl
from jax.experimental.pallas import tpu as pltpu

_DILATIONS = (12, 24, 36)
_CH = 8


def _fused_kernel(x_ref, b0w_ref, b0o_ref, dilw_ref, dilo_ref,
                  poolw_ref, poolo_ref, projw_ref, projo_ref,
                  headw_ref, heado_ref, clsw_ref, clsb_ref,
                  o_ref, xps, pbuf, *, H, W, P, dils):
    cin = x_ref.shape[1]
    C = b0w_ref.shape[-1]
    CP = clsw_ref.shape[-1]
    f32 = jnp.float32
    bf16 = jnp.bfloat16
    CH = _CH
    M = CH * W

    xps[...] = jnp.zeros_like(xps)
    for c in range(H // CH):
        xc = x_ref[0, :, c * M:(c + 1) * M].astype(bf16)
        xct = jnp.transpose(xc, (1, 0))
        xps[P + c * CH:P + c * CH + CH, P:P + W, :] = xct.reshape(CH, W, cin)

    interior = xps[P:P + H, P:P + W, :].reshape(H * W, cin)
    mean = jnp.mean(interior.astype(f32), axis=0, keepdims=True)
    pooled = jnp.dot(mean.astype(bf16), poolw_ref[...],
                     preferred_element_type=f32)
    pooled = jnp.maximum(pooled + poolo_ref[...], 0.0)
    pool_proj = jnp.dot(pooled.astype(bf16), projw_ref[4],
                        preferred_element_type=f32)

    pbuf[0:1, :, :] = jnp.zeros((1, W + 2, C), bf16)
    pbuf[H + 1:H + 2, :, :] = jnp.zeros((1, W + 2, C), bf16)
    pbuf[:, 0:1, :] = jnp.zeros((H + 2, 1, C), bf16)
    pbuf[:, W + 1:W + 2, :] = jnp.zeros((H + 2, 1, C), bf16)

    for c in range(H // CH):
        r0 = c * CH
        xs = xps[P + r0:P + r0 + CH, P:P + W, :].reshape(M, cin)
        b0 = jnp.dot(xs, b0w_ref[...], preferred_element_type=f32)
        b0 = jnp.maximum(b0 + b0o_ref[...], 0.0)
        pacc = jnp.dot(b0.astype(bf16), projw_ref[0],
                       preferred_element_type=f32) + pool_proj
        for i, d in enumerate(dils):
            conv = None
            for kh in range(3):
                dh = (kh - 1) * d
                lo = max(0, -dh)
                hi = H - max(0, dh)
                if r0 + CH <= lo or r0 >= hi:
                    continue
                for kw in range(3):
                    dw = (kw - 1) * d
                    patch = xps[P + dh + r0:P + dh + r0 + CH,
                                P + dw:P + dw + W, :].reshape(M, cin)
                    t = jnp.dot(patch, dilw_ref[i * 9 + kh * 3 + kw],
                                preferred_element_type=f32)
                    conv = t if conv is None else conv + t
            bi = jnp.maximum(conv + dilo_ref[i], 0.0)
            pacc = pacc + jnp.dot(bi.astype(bf16), projw_ref[i + 1],
                                  preferred_element_type=f32)
        proj = jnp.maximum(pacc + projo_ref[...], 0.0).astype(bf16)
        pbuf[1 + r0:1 + r0 + CH, 1:W + 1, :] = proj.reshape(CH, W, C)

    for c in range(H // CH):
        r0 = c * CH
        hacc = None
        for kh in range(3):
            for kw in range(3):
                patch = pbuf[r0 + kh:r0 + kh + CH, kw:kw + W, :].reshape(M, C)
                t = jnp.dot(patch, headw_ref[kh * 3 + kw],
                            preferred_element_type=f32)
                hacc = t if hacc is None else hacc + t
        h = jnp.maximum(hacc + heado_ref[...], 0.0).astype(bf16)
        logits = jax.lax.dot_general(clsw_ref[...], h,
                                     (((0,), (1,)), ((), ())),
                                     preferred_element_type=f32)
        o_ref[0, :, r0 * W:r0 * W + M] = logits + clsb_ref[...]


def kernel(b0_w, b0_scale, b0_offset, dil_w, dil_scale, dil_offset,
           pool_w, pool_scale, pool_offset, proj_w, proj_scale, proj_offset,
           head_w, head_scale, head_offset, cls_w, cls_b, x):
    N, cin, H, W = x.shape
    C = b0_w.shape[-1]
    P = max(_DILATIONS)
    nc = cls_w.shape[1]
    CP = max(32, ((nc + 7) // 8) * 8)
    HW = H * W
    bf = jnp.bfloat16

    x2 = x.reshape(N, cin, HW)
    Hp, Wp = H + 2 * P, W + 2 * P

    b0w = (b0_w * b0_scale).astype(bf)
    dilw = (dil_w * dil_scale[:, None, None]).reshape(9 * len(_DILATIONS),
                                                     cin, C).astype(bf)
    poolw = (pool_w * pool_scale).astype(bf)
    projw = (proj_w * proj_scale).astype(bf)
    headw = (head_w * head_scale).reshape(9, C, C).astype(bf)
    clsw = jnp.pad(cls_w, ((0, 0), (0, CP - nc))).astype(bf)
    clsb = jnp.pad(cls_b, ((0, 0), (0, CP - nc))).reshape(CP, 1)

    def const(*shape):
        nd = len(shape)
        return pl.BlockSpec(shape, lambda n, _nd=nd: (0,) * _nd)

    out = pl.pallas_call(
        functools.partial(_fused_kernel, H=H, W=W, P=P, dils=_DILATIONS),
        out_shape=jax.ShapeDtypeStruct((N, CP, HW), jnp.float32),
        grid=(N,),
        in_specs=[
            pl.BlockSpec((1, cin, HW), lambda n: (n, 0, 0)),
            const(cin, C), const(1, C),
            const(9 * len(_DILATIONS), cin, C), const(len(_DILATIONS), 1, C),
            const(cin, C), const(1, C),
            const(5, C, C), const(1, C),
            const(9, C, C), const(1, C),
            const(cin, CP), const(CP, 1),
        ],
        out_specs=pl.BlockSpec((1, CP, HW), lambda n: (n, 0, 0)),
        scratch_shapes=[
            pltpu.VMEM((Hp, Wp, cin), bf),
            pltpu.VMEM((H + 2, W + 2, C), bf),
        ],
        compiler_params=pltpu.CompilerParams(
            dimension_semantics=("parallel",),
            vmem_limit_bytes=100 * 1024 * 1024),
    )(x2, b0w, b0_offset, dilw, dil_offset, poolw, pool_offset,
      projw, proj_offset, headw, head_offset, clsw, clsb)
    return out[:, :nc, :].reshape(N, nc, H, W)


# speedup vs baseline: 1.1459x; 1.0091x over previous
import functools

import jax
import jax.numpy as jnp
from jax.experimental import pallas as pl
from jax.experimental.pallas import tpu as pltpu

_DILATIONS = (12, 24, 36)
_CH = 8


def _fused_kernel(x_ref, b0w_ref, b0o_ref, dilw_ref, dilo_ref,
                  poolw_ref, poolo_ref, projw_ref, projo_ref,
                  headw_ref, heado_ref, clsw_ref, clsb_ref,
                  o_ref, xpa, xpb, pb0, pb1, pb2, *, H, W, P, dils):
    cin = x_ref.shape[1]
    C = b0w_ref.shape[-1]
    CP = clsw_ref.shape[-1]
    f32 = jnp.float32
    bf16 = jnp.bfloat16
    CH = _CH
    M = CH * W
    QA, QB = 40, 44

    xpa[...] = jnp.zeros_like(xpa)
    xpb[...] = jnp.zeros_like(xpb)
    for c in range(H // CH):
        xc = x_ref[0, :, c * M:(c + 1) * M].astype(bf16)
        xct = jnp.transpose(xc, (1, 0)).reshape(CH, W, cin)
        xpa[P + c * CH:P + c * CH + CH, QA:QA + W, :] = xct
        xpb[P + c * CH:P + c * CH + CH, QB:QB + W, :] = xct

    interior = xpa[P:P + H, QA:QA + W, :].reshape(H * W, cin)
    mean = jnp.mean(interior.astype(f32), axis=0, keepdims=True)
    pooled = jnp.dot(mean.astype(bf16), poolw_ref[...],
                     preferred_element_type=f32)
    pooled = jnp.maximum(pooled + poolo_ref[...], 0.0)
    pool_proj = jnp.dot(pooled.astype(bf16), projw_ref[4],
                        preferred_element_type=f32)

    pb0[...] = jnp.zeros_like(pb0)
    pb1[...] = jnp.zeros_like(pb1)
    pb2[...] = jnp.zeros_like(pb2)

    for c in range(H // CH):
        r0 = c * CH
        xs = xpa[P + r0:P + r0 + CH, QA:QA + W, :].reshape(M, cin)
        b0 = jnp.dot(xs, b0w_ref[...], preferred_element_type=f32)
        b0 = jnp.maximum(b0 + b0o_ref[...], 0.0)
        pacc = jnp.dot(b0.astype(bf16), projw_ref[0],
                       preferred_element_type=f32) + pool_proj
        for i, d in enumerate(dils):
            conv = None
            for kh in range(3):
                dh = (kh - 1) * d
                lo = max(0, -dh)
                hi = H - max(0, dh)
                if r0 + CH <= lo or r0 >= hi:
                    continue
                for kw in range(3):
                    dw = (kw - 1) * d
                    if (QA + dw) % 8 == 0:
                        patch = xpa[P + dh + r0:P + dh + r0 + CH,
                                    QA + dw:QA + dw + W, :]
                    else:
                        patch = xpb[P + dh + r0:P + dh + r0 + CH,
                                    QB + dw:QB + dw + W, :]
                    patch = patch.reshape(M, cin)
                    t = jnp.dot(patch, dilw_ref[i * 9 + kh * 3 + kw],
                                preferred_element_type=f32)
                    conv = t if conv is None else conv + t
            bi = jnp.maximum(conv + dilo_ref[i], 0.0)
            pacc = pacc + jnp.dot(bi.astype(bf16), projw_ref[i + 1],
                                  preferred_element_type=f32)
        proj = jnp.maximum(pacc + projo_ref[...], 0.0).astype(bf16)
        pr = proj.reshape(CH, W, C)
        pb1[1 + r0:1 + r0 + CH, :, :] = pr
        pb0[1 + r0:1 + r0 + CH, 1:W, :] = pr[:, :W - 1, :]
        pb2[1 + r0:1 + r0 + CH, 0:W - 1, :] = pr[:, 1:, :]

    for c in range(H // CH):
        r0 = c * CH
        hacc = None
        for kh in range(3):
            for kw, pb in enumerate((pb0, pb1, pb2)):
                patch = pb[r0 + kh:r0 + kh + CH, :, :].reshape(M, C)
                t = jnp.dot(patch, headw_ref[kh * 3 + kw],
                            preferred_element_type=f32)
                hacc = t if hacc is None else hacc + t
        h = jnp.maximum(hacc + heado_ref[...], 0.0).astype(bf16)
        logits = jax.lax.dot_general(clsw_ref[...], h,
                                     (((0,), (1,)), ((), ())),
                                     preferred_element_type=f32)
        o_ref[0, :, r0 * W:r0 * W + M] = logits + clsb_ref[...]


def kernel(b0_w, b0_scale, b0_offset, dil_w, dil_scale, dil_offset,
           pool_w, pool_scale, pool_offset, proj_w, proj_scale, proj_offset,
           head_w, head_scale, head_offset, cls_w, cls_b, x):
    N, cin, H, W = x.shape
    C = b0_w.shape[-1]
    P = max(_DILATIONS)
    nc = cls_w.shape[1]
    CP = max(32, ((nc + 7) // 8) * 8)
    HW = H * W
    bf = jnp.bfloat16

    x2 = x.reshape(N, cin, HW)
    Hp, Wp = H + 2 * P, W + 2 * P

    b0w = (b0_w * b0_scale).astype(bf)
    dilw = (dil_w * dil_scale[:, None, None]).reshape(9 * len(_DILATIONS),
                                                     cin, C).astype(bf)
    poolw = (pool_w * pool_scale).astype(bf)
    projw = (proj_w * proj_scale).astype(bf)
    headw = (head_w * head_scale).reshape(9, C, C).astype(bf)
    clsw = jnp.pad(cls_w, ((0, 0), (0, CP - nc))).astype(bf)
    clsb = jnp.pad(cls_b, ((0, 0), (0, CP - nc))).reshape(CP, 1)

    def const(*shape):
        nd = len(shape)
        return pl.BlockSpec(shape, lambda n, _nd=nd: (0,) * _nd)

    out = pl.pallas_call(
        functools.partial(_fused_kernel, H=H, W=W, P=P, dils=_DILATIONS),
        out_shape=jax.ShapeDtypeStruct((N, CP, HW), jnp.float32),
        grid=(N,),
        in_specs=[
            pl.BlockSpec((1, cin, HW), lambda n: (n, 0, 0)),
            const(cin, C), const(1, C),
            const(9 * len(_DILATIONS), cin, C), const(len(_DILATIONS), 1, C),
            const(cin, C), const(1, C),
            const(5, C, C), const(1, C),
            const(9, C, C), const(1, C),
            const(cin, CP), const(CP, 1),
        ],
        out_specs=pl.BlockSpec((1, CP, HW), lambda n: (n, 0, 0)),
        scratch_shapes=[
            pltpu.VMEM((Hp, 40 + W + P, cin), bf),
            pltpu.VMEM((Hp, 44 + W + P, cin), bf),
            pltpu.VMEM((H + 2, W, C), bf),
            pltpu.VMEM((H + 2, W, C), bf),
            pltpu.VMEM((H + 2, W, C), bf),
        ],
        compiler_params=pltpu.CompilerParams(
            dimension_semantics=("parallel",),
            vmem_limit_bytes=100 * 1024 * 1024),
    )(x2, b0w, b0_offset, dilw, dil_offset, poolw, pool_offset,
      projw, proj_offset, headw, head_offset, clsw, clsb)
    return out[:, :nc, :].reshape(N, nc, H, W)


# speedup vs baseline: 1.2789x; 1.1161x over previous
import functools

import jax
import jax.numpy as jnp
from jax.experimental import pallas as pl
from jax.experimental.pallas import tpu as pltpu

_DILATIONS = (12, 24, 36)
_CH = 8


def _fused_kernel(x_ref, b0w_ref, b0o_ref, dilw_ref, dilo_ref,
                  poolw_ref, poolo_ref, projw_ref, projo_ref,
                  headw_ref, heado_ref, clsw_ref, clsb_ref,
                  o_ref, xpa, xpb, pb0, pb1, pb2, *, H, W, P, dils):
    cin = x_ref.shape[-1]
    C = b0w_ref.shape[-1]
    CP = clsw_ref.shape[-1]
    f32 = jnp.float32
    bf16 = jnp.bfloat16
    CH = _CH
    M = CH * W
    QA, QB = 40, 44

    xpa[...] = jnp.zeros_like(xpa)
    xpb[...] = jnp.zeros_like(xpb)
    xi = x_ref[0]
    xpa[P:P + H, QA:QA + W, :] = xi
    xpb[P:P + H, QB:QB + W, :] = xi

    interior = xpa[P:P + H, QA:QA + W, :].reshape(H * W, cin)
    ones = jnp.ones((8, H * W), bf16)
    mean = (jnp.dot(ones, interior, preferred_element_type=f32)[0:1]
            * (1.0 / (H * W)))
    pooled = jnp.dot(mean.astype(bf16), poolw_ref[...],
                     preferred_element_type=f32)
    pooled = jnp.maximum(pooled + poolo_ref[...], 0.0)
    pool_proj = jnp.dot(pooled.astype(bf16), projw_ref[4],
                        preferred_element_type=f32)

    pb0[...] = jnp.zeros_like(pb0)
    pb1[...] = jnp.zeros_like(pb1)
    pb2[...] = jnp.zeros_like(pb2)

    for c in range(H // CH):
        r0 = c * CH
        xs = xpa[P + r0:P + r0 + CH, QA:QA + W, :].reshape(M, cin)
        b0 = jnp.dot(xs, b0w_ref[...], preferred_element_type=f32)
        b0 = jnp.maximum(b0 + b0o_ref[...], 0.0)
        pacc = jnp.dot(b0.astype(bf16), projw_ref[0],
                       preferred_element_type=f32) + pool_proj
        for i, d in enumerate(dils):
            conv = None
            for kh in range(3):
                dh = (kh - 1) * d
                lo = max(0, -dh)
                hi = H - max(0, dh)
                if r0 + CH <= lo or r0 >= hi:
                    continue
                for kw in range(3):
                    dw = (kw - 1) * d
                    if (QA + dw) % 8 == 0:
                        patch = xpa[P + dh + r0:P + dh + r0 + CH,
                                    QA + dw:QA + dw + W, :]
                    else:
                        patch = xpb[P + dh + r0:P + dh + r0 + CH,
                                    QB + dw:QB + dw + W, :]
                    patch = patch.reshape(M, cin)
                    t = jnp.dot(patch, dilw_ref[i * 9 + kh * 3 + kw],
                                preferred_element_type=f32)
                    conv = t if conv is None else conv + t
            bi = jnp.maximum(conv + dilo_ref[i], 0.0)
            pacc = pacc + jnp.dot(bi.astype(bf16), projw_ref[i + 1],
                                  preferred_element_type=f32)
        proj = jnp.maximum(pacc + projo_ref[...], 0.0).astype(bf16)
        pr = proj.reshape(CH, W, C)
        pb1[1 + r0:1 + r0 + CH, :, :] = pr
        pb0[1 + r0:1 + r0 + CH, 1:W, :] = pr[:, :W - 1, :]
        pb2[1 + r0:1 + r0 + CH, 0:W - 1, :] = pr[:, 1:, :]

    for c in range(H // CH):
        r0 = c * CH
        hacc = None
        for kh in range(3):
            for kw, pb in enumerate((pb0, pb1, pb2)):
                patch = pb[r0 + kh:r0 + kh + CH, :, :].reshape(M, C)
                t = jnp.dot(patch, headw_ref[kh * 3 + kw],
                            preferred_element_type=f32)
                hacc = t if hacc is None else hacc + t
        h = jnp.maximum(hacc + heado_ref[...], 0.0).astype(bf16)
        logits = jax.lax.dot_general(clsw_ref[...], h,
                                     (((0,), (1,)), ((), ())),
                                     preferred_element_type=f32)
        nco = o_ref.shape[1]
        o_ref[0, :, r0 * W:r0 * W + M] = (logits + clsb_ref[...])[:nco]


def kernel(b0_w, b0_scale, b0_offset, dil_w, dil_scale, dil_offset,
           pool_w, pool_scale, pool_offset, proj_w, proj_scale, proj_offset,
           head_w, head_scale, head_offset, cls_w, cls_b, x):
    N, cin, H, W = x.shape
    C = b0_w.shape[-1]
    P = max(_DILATIONS)
    nc = cls_w.shape[1]
    CP = max(32, ((nc + 7) // 8) * 8)
    HW = H * W
    bf = jnp.bfloat16

    xh = jnp.transpose(x, (0, 2, 3, 1)).astype(bf)
    Hp, Wp = H + 2 * P, W + 2 * P

    b0w = (b0_w * b0_scale).astype(bf)
    dilw = (dil_w * dil_scale[:, None, None]).reshape(9 * len(_DILATIONS),
                                                     cin, C).astype(bf)
    poolw = (pool_w * pool_scale).astype(bf)
    projw = (proj_w * proj_scale).astype(bf)
    headw = (head_w * head_scale).reshape(9, C, C).astype(bf)
    clsw = jnp.pad(cls_w, ((0, 0), (0, CP - nc))).astype(bf)
    clsb = jnp.pad(cls_b, ((0, 0), (0, CP - nc))).reshape(CP, 1)

    def const(*shape):
        nd = len(shape)
        return pl.BlockSpec(shape, lambda n, _nd=nd: (0,) * _nd)

    out = pl.pallas_call(
        functools.partial(_fused_kernel, H=H, W=W, P=P, dils=_DILATIONS),
        out_shape=jax.ShapeDtypeStruct((N, nc, HW), jnp.float32),
        grid=(N,),
        in_specs=[
            pl.BlockSpec((1, H, W, cin), lambda n: (n, 0, 0, 0)),
            const(cin, C), const(1, C),
            const(9 * len(_DILATIONS), cin, C), const(len(_DILATIONS), 1, C),
            const(cin, C), const(1, C),
            const(5, C, C), const(1, C),
            const(9, C, C), const(1, C),
            const(cin, CP), const(CP, 1),
        ],
        out_specs=pl.BlockSpec((1, nc, HW), lambda n: (n, 0, 0)),
        scratch_shapes=[
            pltpu.VMEM((Hp, 40 + W + P, cin), bf),
            pltpu.VMEM((Hp, 44 + W + P, cin), bf),
            pltpu.VMEM((H + 2, W, C), bf),
            pltpu.VMEM((H + 2, W, C), bf),
            pltpu.VMEM((H + 2, W, C), bf),
        ],
        compiler_params=pltpu.CompilerParams(
            dimension_semantics=("parallel",),
            vmem_limit_bytes=100 * 1024 * 1024),
    )(xh, b0w, b0_offset, dilw, dil_offset, poolw, pool_offset,
      projw, proj_offset, headw, head_offset, clsw, clsb)
    return out.reshape(N, nc, H, W)
